# baseline (speedup 1.0000x reference)
import functools

import jax
import jax.numpy as jnp
from jax import lax
from jax.experimental import pallas as pl
from jax.experimental.pallas import tpu as pltpu
from jax.experimental.pallas import tpu_sc as plsc

NU = 10000
NM = 10000
H = 256
HH = 128
E = 160000
L = 16
NC = 2
NS = 16
NW = NC * NS

EPT = E // NS
ECH = 80
ENK = EPT // ECH
CHR = 80
NCHR = NU // CHR

EP = 163840
TPT = EP // NS
DCH = 64
DNK = TPT // DCH


@functools.lru_cache(maxsize=1)
def _mesh():
    return plsc.VectorSubcoreMesh(core_axis_name="c", subcore_axis_name="s",
                                  num_cores=NC, num_subcores=NS)


def _row_chunks(s, body):
    def j_loop(j, _):
        m = s + NS * j

        @pl.when(m < NCHR)
        def _():
            body(m * CHR)

        return 0

    lax.fori_loop(0, (NCHR + NS - 1) // NS, j_loop, 0)


def _deg_body(src_hbm, dst_hbm, rec_u_hbm, rec_m_hbm,
              acc, ixall, ones_v, fbuf, obuf):
    c = lax.axis_index("c")
    s = lax.axis_index("s")

    for j in range(ECH // L):
        ones_v[pl.ds(j * L, L)] = jnp.ones((L,), jnp.float32)
        fbuf[pl.ds(j * L, L)] = jnp.zeros((L,), jnp.float32)

    _row_chunks(s, lambda r0: pltpu.sync_copy(fbuf, acc.at[pl.ds(r0, CHR)]))

    def run(idx_hbm, out_hbm):
        pltpu.sync_copy(idx_hbm.at[s], ixall)
        plsc.subcore_barrier()

        def step(k, _):
            pltpu.sync_copy(ones_v, acc.at[ixall.at[k, 0]], add=True)
            return 0

        lax.fori_loop(0, ENK, step, 0)
        plsc.subcore_barrier()

        def sub(r0):
            pltpu.sync_copy(acc.at[pl.ds(r0, CHR)], fbuf)
            for j in range(CHR // L):
                v = fbuf[pl.ds(j * L, L)]
                obuf[pl.ds(j * L, L)] = 1.0 / jnp.maximum(v, 1.0)
            pltpu.sync_copy(obuf, out_hbm.at[pl.ds(r0, CHR)])

        _row_chunks(s, sub)

    @pl.when(c == 0)
    def _():
        run(src_hbm, rec_u_hbm)

    @pl.when(c == 1)
    def _():
        run(dst_hbm, rec_m_hbm)


def _sc_degrees(src4, dst4):
    k = pl.kernel(
        _deg_body,
        out_type=(jax.ShapeDtypeStruct((NU,), jnp.float32),
                  jax.ShapeDtypeStruct((NM,), jnp.float32)),
        mesh=_mesh(),
        scratch_types=[
            pltpu.VMEM_SHARED((NU,), jnp.float32),
            pltpu.VMEM((ENK, 1, ECH), jnp.int32),
            pltpu.VMEM((ECH,), jnp.float32),
            pltpu.VMEM((CHR,), jnp.float32),
            pltpu.VMEM((CHR,), jnp.float32),
        ],
    )
    return k(src4, dst4)


NBA = 3


def _agg_body(tlo, thi, gidx_hbm, sidx_hbm, olo, ohi,
              acc, gb0, gb1, gb2, sb0, sb1, sb2, r0buf, r1buf, r2buf,
              semr0, semr1, semr2, semg0, semg1, semg2,
              sems0, sems1, sems2):
    c = lax.axis_index("c")
    s = lax.axis_index("s")
    rows = [r0buf, r1buf, r2buf]
    gbufs = [gb0, gb1, gb2]
    sbufs = [sb0, sb1, sb2]
    semr = [semr0, semr1, semr2]
    semg = [semg0, semg1, semg2]
    sems = [sems0, sems1, sems2]

    def zrow(i, _):
        for jj in range(HH // L):
            r0buf[i, pl.ds(jj * L, L)] = jnp.zeros((L,), jnp.float32)
        return 0

    lax.fori_loop(0, CHR, zrow, 0)
    _row_chunks(s, lambda r0: pltpu.sync_copy(r0buf, acc.at[pl.ds(r0, CHR), :]))
    plsc.subcore_barrier()

    def run(tbl, out_hbm):
        for b in range(NBA):
            pltpu.sync_copy(gidx_hbm.at[s, b, 0], gbufs[b])
            pltpu.async_copy(sidx_hbm.at[s, b, 0], sbufs[b], sems[b])
            pltpu.async_copy(tbl.at[gbufs[b]], rows[b], semr[b])

        def step(k, b):
            pltpu.make_async_copy(tbl.at[gbufs[b]], rows[b], semr[b]).wait()
            pltpu.make_async_copy(sidx_hbm.at[s, k, 0],
                                  sbufs[b], sems[b]).wait()
            pltpu.sync_copy(rows[b], acc.at[sbufs[b]], add=True)

            @pl.when(k + NBA < ENK)
            def _():
                pltpu.async_copy(sidx_hbm.at[s, k + NBA, 0], sbufs[b], sems[b])
                pltpu.async_copy(gidx_hbm.at[s, k + NBA, 0], gbufs[b], semg[b])
                pltpu.make_async_copy(gidx_hbm.at[s, k + NBA, 0], gbufs[b],
                                      semg[b]).wait()
                pltpu.async_copy(tbl.at[gbufs[b]], rows[b], semr[b])

        def outer(g, _):
            for b in range(NBA):
                step(g * NBA + b, b)
            return 0

        nmain = ENK // NBA
        lax.fori_loop(0, nmain, outer, 0)
        for kl in range(nmain * NBA, ENK):
            step(kl, kl % NBA)
        plsc.subcore_barrier()

        def sub(r0):
            pltpu.sync_copy(acc.at[pl.ds(r0, CHR), :], r0buf)
            pltpu.sync_copy(r0buf, out_hbm.at[pl.ds(r0, CHR), :])

        _row_chunks(s, sub)

    @pl.when(c == 0)
    def _():
        run(tlo, olo)

    @pl.when(c == 1)
    def _():
        run(thi, ohi)


def _sc_segsum(tlo, thi, gidx4, sidx4, n_dst):
    k = pl.kernel(
        _agg_body,
        out_type=(jax.ShapeDtypeStruct((n_dst, HH), jnp.float32),
                  jax.ShapeDtypeStruct((n_dst, HH), jnp.float32)),
        mesh=_mesh(),
        scratch_types=(
            [pltpu.VMEM_SHARED((n_dst, HH), jnp.float32)]
            + [pltpu.VMEM((ECH,), jnp.int32)] * (2 * NBA)
            + [pltpu.VMEM((ECH, HH), jnp.float32)] * NBA
            + [pltpu.SemaphoreType.DMA] * (3 * NBA)
        ),
    )
    return k(tlo, thi, gidx4, sidx4)


def _dot_body(ulo, uhi, mlo, mhi, aidx_hbm, bidx_hbm, out0, out1,
              sh, aix0, aix1, bix0, bix1, ru0, ru1, rm0, rm1, ob0, ob1,
              su0, su1, sm0, sm1, sa0, sa1, sb0, sb1, so0, so1):
    c = lax.axis_index("c")
    s = lax.axis_index("s")
    aixs = [aix0, aix1]
    bixs = [bix0, bix1]
    rus = [ru0, ru1]
    rms = [rm0, rm1]
    obs = [ob0, ob1]
    sus = [su0, su1]
    sms = [sm0, sm1]
    sas = [sa0, sa1]
    sbs = [sb0, sb1]
    sos = [so0, so1]

    def run(utab_hbm, mtab_hbm, out_hbm):
        _row_chunks(s, lambda r0: pltpu.sync_copy(
            utab_hbm.at[pl.ds(r0, CHR), :], sh.at[pl.ds(r0, CHR), :]))
        plsc.subcore_barrier()

        for b in range(2):
            pltpu.sync_copy(aidx_hbm.at[s, b, 0], aixs[b])
            pltpu.sync_copy(bidx_hbm.at[s, b, 0], bixs[b])
            pltpu.async_copy(sh.at[aixs[b]], rus[b], sus[b])
            pltpu.async_copy(mtab_hbm.at[bixs[b]], rms[b], sms[b])

        def outer(g, _):
            for b in range(2):
                k = g * 2 + b
                base = s * TPT + k * DCH

                pltpu.make_async_copy(sh.at[aixs[b]], rus[b], sus[b]).wait()
                pltpu.make_async_copy(mtab_hbm.at[bixs[b]], rms[b],
                                      sms[b]).wait()

                @pl.when(k + 2 < DNK)
                def _():
                    pltpu.async_copy(aidx_hbm.at[s, k + 2, 0], aixs[b], sas[b])
                    pltpu.async_copy(bidx_hbm.at[s, k + 2, 0], bixs[b], sbs[b])

                @pl.when(k >= 2)
                def _():
                    pltpu.make_async_copy(
                        obs[b], out_hbm.at[pl.ds(base - 2 * DCH, DCH), :],
                        sos[b]).wait()

                def quad(q, _):
                    for t in range(4):
                        e = q * 4 + t
                        acc = rus[b][e, pl.ds(0, L)] * rms[b][e, pl.ds(0, L)]
                        for jj in range(1, HH // L):
                            acc += (rus[b][e, pl.ds(jj * L, L)]
                                    * rms[b][e, pl.ds(jj * L, L)])
                        obs[b][e, pl.ds(0, L)] = acc
                    return 0

                lax.fori_loop(0, DCH // 4, quad, 0)
                pltpu.async_copy(obs[b], out_hbm.at[pl.ds(base, DCH), :],
                                 sos[b])

                @pl.when(k + 2 < DNK)
                def _():
                    pltpu.make_async_copy(aidx_hbm.at[s, k + 2, 0], aixs[b],
                                          sas[b]).wait()
                    pltpu.make_async_copy(bidx_hbm.at[s, k + 2, 0], bixs[b],
                                          sbs[b]).wait()
                    pltpu.async_copy(sh.at[aixs[b]], rus[b], sus[b])
                    pltpu.async_copy(mtab_hbm.at[bixs[b]], rms[b], sms[b])

            return 0

        lax.fori_loop(0, DNK // 2, outer, 0)
        for b in range(2):
            kl = DNK - 2 + b
            base = s * TPT + kl * DCH
            pltpu.make_async_copy(obs[b], out_hbm.at[pl.ds(base, DCH), :],
                                  sos[b]).wait()

    @pl.when(c == 0)
    def _():
        run(ulo, mlo, out0)

    @pl.when(c == 1)
    def _():
        run(uhi, mhi, out1)


def _sc_dot(ulo, uhi, mlo, mhi, aidx4, bidx4):
    k = pl.kernel(
        _dot_body,
        out_type=(jax.ShapeDtypeStruct((EP, L), jnp.float32),
                  jax.ShapeDtypeStruct((EP, L), jnp.float32)),
        mesh=_mesh(),
        scratch_types=(
            [pltpu.VMEM_SHARED((NU, HH), jnp.float32)]
            + [pltpu.VMEM((DCH,), jnp.int32)] * 4
            + [pltpu.VMEM((DCH, HH), jnp.float32)] * 4
            + [pltpu.VMEM((DCH, L), jnp.float32)] * 2
            + [pltpu.SemaphoreType.DMA] * 10
        ),
    )
    return k(ulo, uhi, mlo, mhi, aidx4, bidx4)


FB = 2048


def _fin_body(p0_ref, p1_ref, o_ref):
    p = (p0_ref[...] + p1_ref[...]).reshape(FB, 8, L)
    o_ref[...] = jnp.sum(p, axis=-1)


def _tc_finish(part0, part1):
    p0 = part0.reshape(EP // 8, HH)
    p1 = part1.reshape(EP // 8, HH)
    spec = pl.BlockSpec((FB, HH), lambda i: (i, 0))
    return pl.pallas_call(
        _fin_body,
        grid=(EP // 8 // FB,),
        in_specs=[spec, spec],
        out_specs=pl.BlockSpec((FB, 8), lambda i: (i, 0)),
        out_shape=jax.ShapeDtypeStruct((EP // 8, 8), jnp.float32),
    )(p0, p1).reshape(EP)


RB = 1000


def _tc_body(relu, split_out,
             amlo, amhi, recm, xmlo, xmhi, w_um_lt, w_um_rt, b_um,
             aulo, auhi, recu, xulo, xuhi, w_mu_lt, w_mu_rt, b_mu,
             *outs):
    def side(mlo, mhi, rec, xlo, xhi, wlt, wrt, b):
        r = rec[...]
        y = jnp.dot(mlo[...] * r, wlt[pl.ds(0, HH), :],
                    preferred_element_type=jnp.float32)
        y += jnp.dot(mhi[...] * r, wlt[pl.ds(HH, HH), :],
                     preferred_element_type=jnp.float32)
        y += jnp.dot(xlo[...], wrt[pl.ds(0, HH), :],
                     preferred_element_type=jnp.float32)
        y += jnp.dot(xhi[...], wrt[pl.ds(HH, HH), :],
                     preferred_element_type=jnp.float32)
        y += b[...]
        if relu:
            y = jnp.maximum(y, 0.0)
        return y

    ym = side(amlo, amhi, recm, xmlo, xmhi, w_um_lt, w_um_rt, b_um)
    yu = side(aulo, auhi, recu, xulo, xuhi, w_mu_lt, w_mu_rt, b_mu)
    if split_out:
        outs[0][...] = ym[:, :HH]
        outs[1][...] = ym[:, HH:]
        outs[2][...] = yu[:, :HH]
        outs[3][...] = yu[:, HH:]
    else:
        outs[0][...] = ym
        outs[1][...] = yu


def _tc_layer(amlo, amhi, recm, xmlo, xmhi, wl_um, wr_um, b_um,
              aulo, auhi, recu, xulo, xuhi, wl_mu, wr_mu, b_mu,
              relu, split_out):
    half = pl.BlockSpec((RB, HH), lambda i: (i, 0))
    rspec = pl.BlockSpec((RB, 1), lambda i: (i, 0))
    wspec = pl.BlockSpec((H, H), lambda i: (0, 0))
    bspec = pl.BlockSpec((1, H), lambda i: (0, 0))
    in_specs = [half, half, rspec, half, half, wspec, wspec, bspec,
                half, half, rspec, half, half, wspec, wspec, bspec]
    if split_out:
        out_shape = tuple(jax.ShapeDtypeStruct((NU, HH), jnp.float32)
                          for _ in range(4))
        out_specs = (half, half, half, half)
    else:
        full = pl.BlockSpec((RB, H), lambda i: (i, 0))
        out_shape = tuple(jax.ShapeDtypeStruct((NU, H), jnp.float32)
                          for _ in range(2))
        out_specs = (full, full)
    return pl.pallas_call(
        functools.partial(_tc_body, relu, split_out),
        grid=(NU // RB,),
        in_specs=in_specs,
        out_specs=out_specs,
        out_shape=out_shape,
    )(amlo, amhi, recm, xmlo, xmhi, wl_um.T, wr_um.T, b_um.reshape(1, H),
      aulo, auhi, recu, xulo, xuhi, wl_mu.T, wr_mu.T, b_mu.reshape(1, H))


def kernel(user_emb, movie_emb, W1_um_l, b1_um, W1_um_r, W1_mu_l, b1_mu,
           W1_mu_r, W2_um_l, b2_um, W2_um_r, W2_mu_l, b2_mu, W2_mu_r,
           user_node_id, movie_node_id, edge_index, edge_label_index):
    src4 = edge_index[0].reshape(NS, ENK, 1, ECH)
    dst4 = edge_index[1].reshape(NS, ENK, 1, ECH)

    xu_lo, xu_hi = user_emb[:, :HH], user_emb[:, HH:]
    xm_lo, xm_hi = movie_emb[:, :HH], movie_emb[:, HH:]

    rec_u, rec_m = _sc_degrees(src4, dst4)
    rec_u = rec_u.reshape(NU, 1)
    rec_m = rec_m.reshape(NM, 1)

    am_lo, am_hi = _sc_segsum(xu_lo, xu_hi, src4, dst4, NM)
    au_lo, au_hi = _sc_segsum(xm_lo, xm_hi, dst4, src4, NU)

    m1_lo, m1_hi, u1_lo, u1_hi = _tc_layer(
        am_lo, am_hi, rec_m, xm_lo, xm_hi, W1_um_l, W1_um_r, b1_um,
        au_lo, au_hi, rec_u, xu_lo, xu_hi, W1_mu_l, W1_mu_r, b1_mu,
        relu=True, split_out=True)

    am2_lo, am2_hi = _sc_segsum(u1_lo, u1_hi, src4, dst4, NM)
    au2_lo, au2_hi = _sc_segsum(m1_lo, m1_hi, dst4, src4, NU)

    m2_lo, m2_hi, u2_lo, u2_hi = _tc_layer(
        am2_lo, am2_hi, rec_m, m1_lo, m1_hi, W2_um_l, W2_um_r, b2_um,
        au2_lo, au2_hi, rec_u, u1_lo, u1_hi, W2_mu_l, W2_mu_r, b2_mu,
        relu=False, split_out=True)

    pad = jnp.zeros((EP - E,), jnp.int32)
    aidx4 = jnp.concatenate([edge_label_index[0], pad]).reshape(NS, DNK, 1, DCH)
    bidx4 = jnp.concatenate([edge_label_index[1], pad]).reshape(NS, DNK, 1, DCH)
    part0, part1 = _sc_dot(u2_lo, u2_hi, m2_lo, m2_hi, aidx4, bidx4)
    scores = _tc_finish(part0, part1)
    return scores[:E]


# speedup vs baseline: 1.1133x; 1.1133x over previous
import functools

import jax
import jax.numpy as jnp
from jax import lax
from jax.experimental import pallas as pl
from jax.experimental.pallas import tpu as pltpu
from jax.experimental.pallas import tpu_sc as plsc

NU = 10000
NM = 10000
H = 256
HH = 128
E = 160000
L = 16
NC = 2
NS = 16
NW = NC * NS

EPT = E // NS
ECH = 80
ENK = EPT // ECH
CHR = 80
NCHR = NU // CHR

TPT = 5120
EP = NW * TPT
DCH = 80
DNK = TPT // DCH


@functools.lru_cache(maxsize=1)
def _mesh():
    return plsc.VectorSubcoreMesh(core_axis_name="c", subcore_axis_name="s",
                                  num_cores=NC, num_subcores=NS)


def _row_chunks(s, body):
    def j_loop(j, _):
        m = s + NS * j

        @pl.when(m < NCHR)
        def _():
            body(m * CHR)

        return 0

    lax.fori_loop(0, (NCHR + NS - 1) // NS, j_loop, 0)


def _deg_body(src_hbm, dst_hbm, rec_u_hbm, rec_m_hbm,
              acc, ixall, ones_v, fbuf, obuf):
    c = lax.axis_index("c")
    s = lax.axis_index("s")

    for j in range(ECH // L):
        ones_v[pl.ds(j * L, L)] = jnp.ones((L,), jnp.float32)
        fbuf[pl.ds(j * L, L)] = jnp.zeros((L,), jnp.float32)

    _row_chunks(s, lambda r0: pltpu.sync_copy(fbuf, acc.at[pl.ds(r0, CHR)]))

    def run(idx_hbm, out_hbm):
        pltpu.sync_copy(idx_hbm.at[s], ixall)
        plsc.subcore_barrier()

        def step(k, _):
            pltpu.sync_copy(ones_v, acc.at[ixall.at[k, 0]], add=True)
            return 0

        lax.fori_loop(0, ENK, step, 0)
        plsc.subcore_barrier()

        def sub(r0):
            pltpu.sync_copy(acc.at[pl.ds(r0, CHR)], fbuf)
            for j in range(CHR // L):
                v = fbuf[pl.ds(j * L, L)]
                obuf[pl.ds(j * L, L)] = 1.0 / jnp.maximum(v, 1.0)
            pltpu.sync_copy(obuf, out_hbm.at[pl.ds(r0, CHR)])

        _row_chunks(s, sub)

    @pl.when(c == 0)
    def _():
        run(src_hbm, rec_u_hbm)

    @pl.when(c == 1)
    def _():
        run(dst_hbm, rec_m_hbm)


def _sc_degrees(src4, dst4):
    k = pl.kernel(
        _deg_body,
        out_type=(jax.ShapeDtypeStruct((NU,), jnp.float32),
                  jax.ShapeDtypeStruct((NM,), jnp.float32)),
        mesh=_mesh(),
        scratch_types=[
            pltpu.VMEM_SHARED((NU,), jnp.float32),
            pltpu.VMEM((ENK, 1, ECH), jnp.int32),
            pltpu.VMEM((ECH,), jnp.float32),
            pltpu.VMEM((CHR,), jnp.float32),
            pltpu.VMEM((CHR,), jnp.float32),
        ],
    )
    return k(src4, dst4)


def _agg_body(tlo, thi, gidx_hbm, sidx_hbm, olo, ohi,
              acc, gixall, sb0, sb1, r0buf, r1buf,
              semr0, semr1, sems0, sems1):
    c = lax.axis_index("c")
    s = lax.axis_index("s")
    rows = [r0buf, r1buf]
    sbufs = [sb0, sb1]
    semr = [semr0, semr1]
    sems = [sems0, sems1]

    def zrow(i, _):
        for jj in range(HH // L):
            r0buf[i, pl.ds(jj * L, L)] = jnp.zeros((L,), jnp.float32)
        return 0

    lax.fori_loop(0, CHR, zrow, 0)
    _row_chunks(s, lambda r0: pltpu.sync_copy(r0buf, acc.at[pl.ds(r0, CHR), :]))

    pltpu.sync_copy(gidx_hbm.at[s], gixall)
    plsc.subcore_barrier()

    def run(tbl, out_hbm):
        for b in range(2):
            pltpu.async_copy(sidx_hbm.at[s, b, 0], sbufs[b], sems[b])
            pltpu.async_copy(tbl.at[gixall.at[b, 0]], rows[b], semr[b])

        def step(k, b):
            pltpu.make_async_copy(tbl.at[gixall.at[k, 0]],
                                  rows[b], semr[b]).wait()
            pltpu.make_async_copy(sidx_hbm.at[s, k, 0],
                                  sbufs[b], sems[b]).wait()
            pltpu.sync_copy(rows[b], acc.at[sbufs[b]], add=True)

            @pl.when(k + 2 < ENK)
            def _():
                pltpu.async_copy(sidx_hbm.at[s, k + 2, 0], sbufs[b], sems[b])
                pltpu.async_copy(tbl.at[gixall.at[k + 2, 0]],
                                 rows[b], semr[b])

        def outer(g, _):
            for b in range(2):
                step(g * 2 + b, b)
            return 0

        lax.fori_loop(0, ENK // 2, outer, 0)
        step(ENK - 1, (ENK - 1) % 2)
        plsc.subcore_barrier()

        def sub(r0):
            pltpu.sync_copy(acc.at[pl.ds(r0, CHR), :], r0buf)
            pltpu.sync_copy(r0buf, out_hbm.at[pl.ds(r0, CHR), :])

        _row_chunks(s, sub)

    @pl.when(c == 0)
    def _():
        run(tlo, olo)

    @pl.when(c == 1)
    def _():
        run(thi, ohi)


def _sc_segsum(tlo, thi, gidx4, sidx4, n_dst):
    k = pl.kernel(
        _agg_body,
        out_type=(jax.ShapeDtypeStruct((n_dst, HH), jnp.float32),
                  jax.ShapeDtypeStruct((n_dst, HH), jnp.float32)),
        mesh=_mesh(),
        scratch_types=(
            [pltpu.VMEM_SHARED((n_dst, HH), jnp.float32),
             pltpu.VMEM((ENK, 1, ECH), jnp.int32)]
            + [pltpu.VMEM((ECH,), jnp.int32)] * 2
            + [pltpu.VMEM((ECH, HH), jnp.float32)] * 2
            + [pltpu.SemaphoreType.DMA] * 4
        ),
    )
    return k(tlo, thi, gidx4, sidx4)


def _dot_body(u2, m2, aidx_hbm, bidx_hbm, out_hbm,
              aixall, bixall, ru0, ru1, rm0, rm1, ob0, ob1,
              sa0, sa1, sb0, sb1, so0, so1):
    c = lax.axis_index("c")
    s = lax.axis_index("s")
    w = s * NC + c
    rus = [ru0, ru1]
    rms = [rm0, rm1]
    obs = [ob0, ob1]
    sas = [sa0, sa1]
    sbs = [sb0, sb1]
    sos = [so0, so1]

    pltpu.sync_copy(aidx_hbm.at[w], aixall)
    pltpu.sync_copy(bidx_hbm.at[w], bixall)
    for b in range(2):
        pltpu.async_copy(u2.at[aixall.at[b, 0]], rus[b], sas[b])
        pltpu.async_copy(m2.at[bixall.at[b, 0]], rms[b], sbs[b])

    def outer(g, _):
        for b in range(2):
            k = g * 2 + b
            base = w * TPT + k * DCH
            pltpu.make_async_copy(u2.at[aixall.at[k, 0]], rus[b], sas[b]).wait()
            pltpu.make_async_copy(m2.at[bixall.at[k, 0]], rms[b], sbs[b]).wait()

            @pl.when(k >= 2)
            def _():
                pltpu.make_async_copy(
                    obs[b], out_hbm.at[pl.ds(base - 2 * DCH, DCH), :],
                    sos[b]).wait()

            def edge(e, _):
                acc = rus[b][e, pl.ds(0, L)] * rms[b][e, pl.ds(0, L)]
                for jj in range(1, H // L):
                    acc += (rus[b][e, pl.ds(jj * L, L)]
                            * rms[b][e, pl.ds(jj * L, L)])
                obs[b][e, pl.ds(0, L)] = acc
                return 0

            lax.fori_loop(0, DCH, edge, 0)
            pltpu.async_copy(obs[b], out_hbm.at[pl.ds(base, DCH), :], sos[b])

            @pl.when(k + 2 < DNK)
            def _():
                pltpu.async_copy(u2.at[aixall.at[k + 2, 0]], rus[b], sas[b])
                pltpu.async_copy(m2.at[bixall.at[k + 2, 0]], rms[b], sbs[b])

        return 0

    lax.fori_loop(0, DNK // 2, outer, 0)
    for b in range(2):
        kl = DNK - 2 + b
        base = w * TPT + kl * DCH
        pltpu.make_async_copy(obs[b], out_hbm.at[pl.ds(base, DCH), :],
                              sos[b]).wait()


def _sc_dot(u2, m2, aidx4, bidx4):
    k = pl.kernel(
        _dot_body,
        out_type=jax.ShapeDtypeStruct((EP, L), jnp.float32),
        mesh=_mesh(),
        scratch_types=[
            pltpu.VMEM((DNK, 1, DCH), jnp.int32),
            pltpu.VMEM((DNK, 1, DCH), jnp.int32),
            pltpu.VMEM((DCH, H), jnp.float32),
            pltpu.VMEM((DCH, H), jnp.float32),
            pltpu.VMEM((DCH, H), jnp.float32),
            pltpu.VMEM((DCH, H), jnp.float32),
            pltpu.VMEM((DCH, L), jnp.float32),
            pltpu.VMEM((DCH, L), jnp.float32),
            pltpu.SemaphoreType.DMA,
            pltpu.SemaphoreType.DMA,
            pltpu.SemaphoreType.DMA,
            pltpu.SemaphoreType.DMA,
            pltpu.SemaphoreType.DMA,
            pltpu.SemaphoreType.DMA,
        ],
    )
    return k(u2, m2, aidx4, bidx4)


FB = 2048


def _fin_body(p_ref, o_ref):
    p = p_ref[...].reshape(FB, 8, L)
    o_ref[...] = jnp.sum(p, axis=-1)


def _tc_finish(partials):
    p = partials.reshape(EP // 8, HH)
    return pl.pallas_call(
        _fin_body,
        grid=(EP // 8 // FB,),
        in_specs=[pl.BlockSpec((FB, HH), lambda i: (i, 0))],
        out_specs=pl.BlockSpec((FB, 8), lambda i: (i, 0)),
        out_shape=jax.ShapeDtypeStruct((EP // 8, 8), jnp.float32),
    )(p).reshape(EP)


RB = 1000


def _tc_body(relu, split_out,
             amlo, amhi, recm, xmlo, xmhi, w_um_lt, w_um_rt, b_um,
             aulo, auhi, recu, xulo, xuhi, w_mu_lt, w_mu_rt, b_mu,
             *outs):
    def side(mlo, mhi, rec, xlo, xhi, wlt, wrt, b):
        r = rec[...]
        y = jnp.dot(mlo[...] * r, wlt[pl.ds(0, HH), :],
                    preferred_element_type=jnp.float32)
        y += jnp.dot(mhi[...] * r, wlt[pl.ds(HH, HH), :],
                     preferred_element_type=jnp.float32)
        y += jnp.dot(xlo[...], wrt[pl.ds(0, HH), :],
                     preferred_element_type=jnp.float32)
        y += jnp.dot(xhi[...], wrt[pl.ds(HH, HH), :],
                     preferred_element_type=jnp.float32)
        y += b[...]
        if relu:
            y = jnp.maximum(y, 0.0)
        return y

    ym = side(amlo, amhi, recm, xmlo, xmhi, w_um_lt, w_um_rt, b_um)
    yu = side(aulo, auhi, recu, xulo, xuhi, w_mu_lt, w_mu_rt, b_mu)
    if split_out:
        outs[0][...] = ym[:, :HH]
        outs[1][...] = ym[:, HH:]
        outs[2][...] = yu[:, :HH]
        outs[3][...] = yu[:, HH:]
    else:
        outs[0][...] = ym
        outs[1][...] = yu


def _tc_layer(amlo, amhi, recm, xmlo, xmhi, wl_um, wr_um, b_um,
              aulo, auhi, recu, xulo, xuhi, wl_mu, wr_mu, b_mu,
              relu, split_out):
    half = pl.BlockSpec((RB, HH), lambda i: (i, 0))
    rspec = pl.BlockSpec((RB, 1), lambda i: (i, 0))
    wspec = pl.BlockSpec((H, H), lambda i: (0, 0))
    bspec = pl.BlockSpec((1, H), lambda i: (0, 0))
    in_specs = [half, half, rspec, half, half, wspec, wspec, bspec,
                half, half, rspec, half, half, wspec, wspec, bspec]
    if split_out:
        out_shape = tuple(jax.ShapeDtypeStruct((NU, HH), jnp.float32)
                          for _ in range(4))
        out_specs = (half, half, half, half)
    else:
        full = pl.BlockSpec((RB, H), lambda i: (i, 0))
        out_shape = tuple(jax.ShapeDtypeStruct((NU, H), jnp.float32)
                          for _ in range(2))
        out_specs = (full, full)
    return pl.pallas_call(
        functools.partial(_tc_body, relu, split_out),
        grid=(NU // RB,),
        in_specs=in_specs,
        out_specs=out_specs,
        out_shape=out_shape,
    )(amlo, amhi, recm, xmlo, xmhi, wl_um.T, wr_um.T, b_um.reshape(1, H),
      aulo, auhi, recu, xulo, xuhi, wl_mu.T, wr_mu.T, b_mu.reshape(1, H))


def kernel(user_emb, movie_emb, W1_um_l, b1_um, W1_um_r, W1_mu_l, b1_mu,
           W1_mu_r, W2_um_l, b2_um, W2_um_r, W2_mu_l, b2_mu, W2_mu_r,
           user_node_id, movie_node_id, edge_index, edge_label_index):
    src4 = edge_index[0].reshape(NS, ENK, 1, ECH)
    dst4 = edge_index[1].reshape(NS, ENK, 1, ECH)

    xu_lo, xu_hi = user_emb[:, :HH], user_emb[:, HH:]
    xm_lo, xm_hi = movie_emb[:, :HH], movie_emb[:, HH:]

    rec_u, rec_m = _sc_degrees(src4, dst4)
    rec_u = rec_u.reshape(NU, 1)
    rec_m = rec_m.reshape(NM, 1)

    am_lo, am_hi = _sc_segsum(xu_lo, xu_hi, src4, dst4, NM)
    au_lo, au_hi = _sc_segsum(xm_lo, xm_hi, dst4, src4, NU)

    m1_lo, m1_hi, u1_lo, u1_hi = _tc_layer(
        am_lo, am_hi, rec_m, xm_lo, xm_hi, W1_um_l, W1_um_r, b1_um,
        au_lo, au_hi, rec_u, xu_lo, xu_hi, W1_mu_l, W1_mu_r, b1_mu,
        relu=True, split_out=True)

    am2_lo, am2_hi = _sc_segsum(u1_lo, u1_hi, src4, dst4, NM)
    au2_lo, au2_hi = _sc_segsum(m1_lo, m1_hi, dst4, src4, NU)

    m2, u2 = _tc_layer(
        am2_lo, am2_hi, rec_m, m1_lo, m1_hi, W2_um_l, W2_um_r, b2_um,
        au2_lo, au2_hi, rec_u, u1_lo, u1_hi, W2_mu_l, W2_mu_r, b2_mu,
        relu=False, split_out=False)

    pad = jnp.zeros((EP - E,), jnp.int32)
    aidx4 = jnp.concatenate([edge_label_index[0], pad]).reshape(NW, DNK, 1, DCH)
    bidx4 = jnp.concatenate([edge_label_index[1], pad]).reshape(NW, DNK, 1, DCH)
    partials = _sc_dot(u2, m2, aidx4, bidx4)
    scores = _tc_finish(partials)
    return scores[:E]
